# baseline (speedup 1.0000x reference)
import functools

import jax
import jax.numpy as jnp
from jax import lax
from jax.experimental import pallas as pl
from jax.experimental.pallas import tpu as pltpu
from jax.experimental.pallas import tpu_sc as plsc

B = 16384
D = 64
NUM_REL = 1000
EPS = 1e-6
NC = 2
NS = 16
NW = NC * NS
RPW = B // NW
GROUPS = RPW // 16


def _sc_body(s_hbm, o_hbm, rel_hbm, tab_hbm, out_hbm,
             idx_v, t_v, s_v, o_v, out_v, sem):
    cid = lax.axis_index("c")
    sid = lax.axis_index("s")
    wid = sid * NC + cid
    base = wid * RPW
    fbase = base * D

    copies = [
        pltpu.async_copy(tab_hbm, t_v, sem),
        pltpu.async_copy(rel_hbm.at[pl.ds(base, RPW)], idx_v, sem),
        pltpu.async_copy(s_hbm.at[pl.ds(fbase, RPW * D)], s_v, sem),
        pltpu.async_copy(o_hbm.at[pl.ds(fbase, RPW * D)], o_v, sem),
    ]
    for c in copies:
        c.wait()

    lanes64 = lax.iota(jnp.int32, 16) * D

    def group(g, carry):
        row0 = g * 16
        rel64 = idx_v[pl.ds(row0, 16)] * D
        rows64 = row0 * D + lanes64

        def dstep(d, acc):
            i1 = rows64 + d
            i2 = rel64 + d
            sv = plsc.load_gather(s_v, [i1])
            ov = plsc.load_gather(o_v, [i1])
            tv = plsc.load_gather(t_v, [i2])
            df = sv + tv - ov + EPS
            return acc + df * df

        acc = lax.fori_loop(0, D, dstep, jnp.zeros((16,), jnp.float32))
        bits = lax.bitcast_convert_type(acc, jnp.int32)
        y = lax.bitcast_convert_type(jnp.int32(0x5F3759DF) - (bits >> 1),
                                     jnp.float32)
        for _ in range(3):
            y = y * (1.5 - 0.5 * acc * y * y)
        out_v[pl.ds(row0, 16)] = acc * y
        return carry

    lax.fori_loop(0, GROUPS, group, 0)
    pltpu.sync_copy(out_v, out_hbm.at[pl.ds(base, RPW)])


_sc_call = functools.partial(
    pl.kernel,
    mesh=plsc.VectorSubcoreMesh(core_axis_name="c", subcore_axis_name="s"),
    out_type=jax.ShapeDtypeStruct((B,), jnp.float32),
    compiler_params=pltpu.CompilerParams(needs_layout_passes=False),
    scratch_types=[
        pltpu.VMEM((RPW,), jnp.int32),
        pltpu.VMEM((NUM_REL * D,), jnp.float32),
        pltpu.VMEM((RPW * D,), jnp.float32),
        pltpu.VMEM((RPW * D,), jnp.float32),
        pltpu.VMEM((RPW,), jnp.float32),
        pltpu.SemaphoreType.DMA,
    ],
)(_sc_body)


def kernel(subject_embeddings, object_embeddings, relations, relation_table):
    return _sc_call(subject_embeddings.reshape(B * D),
                    object_embeddings.reshape(B * D),
                    relations.astype(jnp.int32),
                    relation_table.reshape(NUM_REL * D))


# speedup vs baseline: 1.0518x; 1.0518x over previous
import functools

import jax
import jax.numpy as jnp
from jax import lax
from jax.experimental import pallas as pl
from jax.experimental.pallas import tpu as pltpu
from jax.experimental.pallas import tpu_sc as plsc

B = 16384
D = 64
NUM_REL = 1000
EPS = 1e-6
NC = 2
NS = 16
NW = NC * NS
RPW = B // NW
NCH = 4
CR = RPW // NCH
CW = CR * D
CGROUPS = CR // 16


def _sc_body(s_hbm, o_hbm, rel_hbm, tab_hbm, out_hbm,
             idx_v, t_v, s0, o0, s1, o1, out_v, sem_t, sem0, sem1):
    cid = lax.axis_index("c")
    sid = lax.axis_index("s")
    wid = sid * NC + cid
    base = wid * RPW
    fbase = base * D

    bufs = [(s0, o0, sem0), (s1, o1, sem1)]

    def issue(c):
        sv, ov, sem = bufs[c % 2]
        off = fbase + c * CW
        return (pltpu.async_copy(s_hbm.at[pl.ds(off, CW)], sv, sem),
                pltpu.async_copy(o_hbm.at[pl.ds(off, CW)], ov, sem))

    pending = {0: issue(0), 1: issue(1)}
    head = [pltpu.async_copy(tab_hbm, t_v, sem_t),
            pltpu.async_copy(rel_hbm.at[pl.ds(base, RPW)], idx_v, sem_t)]
    for h in head:
        h.wait()

    lanes64 = lax.iota(jnp.int32, 16) * D

    for c in range(NCH):
        sv_ref, ov_ref, _ = bufs[c % 2]
        for h in pending.pop(c):
            h.wait()

        def group(g, carry, sv_ref=sv_ref, ov_ref=ov_ref, c=c):
            row0 = c * CR + g * 16
            rel64 = idx_v[pl.ds(row0, 16)] * D
            rows64 = g * 16 * D + lanes64

            accs = [jnp.zeros((16,), jnp.float32) for _ in range(4)]
            for d in range(D):
                i1 = rows64 + d
                i2 = rel64 + d
                svv = plsc.load_gather(sv_ref, [i1])
                ovv = plsc.load_gather(ov_ref, [i1])
                tvv = plsc.load_gather(t_v, [i2])
                df = svv + tvv - ovv + EPS
                accs[d % 4] = accs[d % 4] + df * df
            acc = (accs[0] + accs[1]) + (accs[2] + accs[3])
            bits = lax.bitcast_convert_type(acc, jnp.int32)
            y = lax.bitcast_convert_type(
                jnp.int32(0x5F3759DF) - (bits >> 1), jnp.float32)
            for _ in range(3):
                y = y * (1.5 - 0.5 * acc * y * y)
            out_v[pl.ds(row0, 16)] = acc * y
            return carry

        lax.fori_loop(0, CGROUPS, group, 0)
        if c + 2 < NCH:
            pending[c + 2] = issue(c + 2)

    pltpu.sync_copy(out_v, out_hbm.at[pl.ds(base, RPW)])


_sc_call = functools.partial(
    pl.kernel,
    mesh=plsc.VectorSubcoreMesh(core_axis_name="c", subcore_axis_name="s"),
    out_type=jax.ShapeDtypeStruct((B,), jnp.float32),
    compiler_params=pltpu.CompilerParams(needs_layout_passes=False),
    scratch_types=[
        pltpu.VMEM((RPW,), jnp.int32),
        pltpu.VMEM((NUM_REL * D,), jnp.float32),
        pltpu.VMEM((CW,), jnp.float32),
        pltpu.VMEM((CW,), jnp.float32),
        pltpu.VMEM((CW,), jnp.float32),
        pltpu.VMEM((CW,), jnp.float32),
        pltpu.VMEM((RPW,), jnp.float32),
        pltpu.SemaphoreType.DMA,
        pltpu.SemaphoreType.DMA,
        pltpu.SemaphoreType.DMA,
    ],
)(_sc_body)


def kernel(subject_embeddings, object_embeddings, relations, relation_table):
    return _sc_call(subject_embeddings.reshape(B * D),
                    object_embeddings.reshape(B * D),
                    relations.astype(jnp.int32),
                    relation_table.reshape(NUM_REL * D))


# speedup vs baseline: 1.8141x; 1.7248x over previous
import functools

import jax
import jax.numpy as jnp
from jax import lax
from jax.experimental import pallas as pl
from jax.experimental.pallas import tpu as pltpu
from jax.experimental.pallas import tpu_sc as plsc

B = 16384
D = 64
NUM_REL = 1000
EPS = 1e-6
NC = 2
NS = 16
NW = NC * NS
RPW = B // NW
NCH = 4
CR = RPW // NCH
CGROUPS = CR // 16


def _sc_body(s_hbm, o_hbm, rel_hbm, tab_hbm, out_hbm,
             idx_v, s0, o0, t0, s1, o1, t1, out_v, sem_t, sem0, sem1):
    cid = lax.axis_index("c")
    sid = lax.axis_index("s")
    wid = sid * NC + cid
    base = wid * RPW

    pltpu.sync_copy(rel_hbm.at[wid], idx_v)

    bufs = [(s0, o0, t0, sem0), (s1, o1, t1, sem1)]

    def issue(c):
        sv, ov, tv, sem = bufs[c % 2]
        rb = base + c * CR
        return (pltpu.async_copy(s_hbm.at[pl.ds(rb, CR)], sv, sem),
                pltpu.async_copy(o_hbm.at[pl.ds(rb, CR)], ov, sem),
                pltpu.async_copy(tab_hbm.at[idx_v.at[c]], tv, sem))

    pending = {0: issue(0), 1: issue(1)}

    lanes = lax.iota(jnp.int32, 16)
    perms = [lanes ^ sh for sh in (8, 4, 2, 1)]

    for c in range(NCH):
        sv_ref, ov_ref, tv_ref, _ = bufs[c % 2]
        for h in pending.pop(c):
            h.wait()

        def group(g, carry, sv_ref=sv_ref, ov_ref=ov_ref, tv_ref=tv_ref,
                  c=c):
            acc = jnp.zeros((16,), jnp.float32)
            for j in range(16):
                r = g * 16 + j
                p = None
                for k in range(4):
                    sk = sv_ref[r, pl.ds(k * 16, 16)]
                    ok = ov_ref[r, pl.ds(k * 16, 16)]
                    tk = tv_ref[r, pl.ds(k * 16, 16)]
                    df = sk + tk - ok + EPS
                    sq = df * df
                    p = sq if p is None else p + sq
                for pm in perms:
                    p = p + jnp.take_along_axis(p, pm, axis=0)
                acc = jnp.where(lanes == j, p, acc)
            bits = lax.bitcast_convert_type(acc, jnp.int32)
            y = lax.bitcast_convert_type(
                jnp.int32(0x5F3759DF) - (bits >> 1), jnp.float32)
            for _ in range(3):
                y = y * (1.5 - 0.5 * acc * y * y)
            out_v[pl.ds(c * CR + g * 16, 16)] = acc * y
            return carry

        lax.fori_loop(0, CGROUPS, group, 0)
        if c + 2 < NCH:
            pending[c + 2] = issue(c + 2)

    pltpu.sync_copy(out_v, out_hbm.at[pl.ds(base, RPW)])


_sc_call = functools.partial(
    pl.kernel,
    mesh=plsc.VectorSubcoreMesh(core_axis_name="c", subcore_axis_name="s"),
    out_type=jax.ShapeDtypeStruct((B,), jnp.float32),
    compiler_params=pltpu.CompilerParams(needs_layout_passes=False,
                                         use_tc_tiling_on_sc=False),
    scratch_types=[
        pltpu.VMEM((NCH, CR), jnp.int32),
        pltpu.VMEM((CR, D), jnp.float32),
        pltpu.VMEM((CR, D), jnp.float32),
        pltpu.VMEM((CR, D), jnp.float32),
        pltpu.VMEM((CR, D), jnp.float32),
        pltpu.VMEM((CR, D), jnp.float32),
        pltpu.VMEM((CR, D), jnp.float32),
        pltpu.VMEM((RPW,), jnp.float32),
        pltpu.SemaphoreType.DMA,
        pltpu.SemaphoreType.DMA,
        pltpu.SemaphoreType.DMA,
    ],
)(_sc_body)


def kernel(subject_embeddings, object_embeddings, relations, relation_table):
    return _sc_call(subject_embeddings, object_embeddings,
                    relations.astype(jnp.int32).reshape(NW, NCH, CR),
                    relation_table)


# speedup vs baseline: 1.9213x; 1.0591x over previous
import functools

import jax
import jax.numpy as jnp
from jax import lax
from jax.experimental import pallas as pl
from jax.experimental.pallas import tpu as pltpu
from jax.experimental.pallas import tpu_sc as plsc

B = 16384
D = 64
NUM_REL = 1000
EPS = 1e-6
NC = 2
NS = 16
NW = NC * NS
RPW = B // NW
NCH = 4
CR = RPW // NCH
CGROUPS = CR // 16


def _sc_body(s_hbm, o_hbm, rel_hbm, tab_hbm, out_hbm,
             idx_v, s0, o0, t0, s1, o1, t1, out_v, sem_t, sem0, sem1):
    cid = lax.axis_index("c")
    sid = lax.axis_index("s")
    wid = sid * NC + cid
    base = wid * RPW

    pltpu.sync_copy(rel_hbm.at[wid], idx_v)

    bufs = [(s0, o0, t0, sem0), (s1, o1, t1, sem1)]

    def issue(c):
        sv, ov, tv, sem = bufs[c % 2]
        rb = base + c * CR
        return (pltpu.async_copy(s_hbm.at[pl.ds(rb, CR)], sv, sem),
                pltpu.async_copy(o_hbm.at[pl.ds(rb, CR)], ov, sem),
                pltpu.async_copy(tab_hbm.at[idx_v.at[c]], tv, sem))

    pending = {0: issue(0), 1: issue(1)}

    lanes = lax.iota(jnp.int32, 16)
    perms = [lanes ^ sh for sh in (8, 4, 2, 1)]

    for c in range(NCH):
        sv_ref, ov_ref, tv_ref, _ = bufs[c % 2]
        for h in pending.pop(c):
            h.wait()

        @plsc.parallel_loop(0, CGROUPS)
        def group(g, sv_ref=sv_ref, ov_ref=ov_ref, tv_ref=tv_ref, c=c):
            acc = jnp.zeros((16,), jnp.float32)
            for j in range(16):
                r = g * 16 + j
                p = None
                for k in range(4):
                    sk = sv_ref[r, pl.ds(k * 16, 16)]
                    ok = ov_ref[r, pl.ds(k * 16, 16)]
                    tk = tv_ref[r, pl.ds(k * 16, 16)]
                    df = sk + tk - ok + EPS
                    sq = df * df
                    p = sq if p is None else p + sq
                for pm in perms:
                    p = p + jnp.take_along_axis(p, pm, axis=0)
                acc = jnp.where(lanes == j, p, acc)
            bits = lax.bitcast_convert_type(acc, jnp.int32)
            y = lax.bitcast_convert_type(
                jnp.int32(0x5F3759DF) - (bits >> 1), jnp.float32)
            for _ in range(3):
                y = y * (1.5 - 0.5 * acc * y * y)
            out_v[pl.ds(c * CR + g * 16, 16)] = acc * y

        if c + 2 < NCH:
            pending[c + 2] = issue(c + 2)

    pltpu.sync_copy(out_v, out_hbm.at[pl.ds(base, RPW)])


_sc_call = functools.partial(
    pl.kernel,
    mesh=plsc.VectorSubcoreMesh(core_axis_name="c", subcore_axis_name="s"),
    out_type=jax.ShapeDtypeStruct((B,), jnp.float32),
    compiler_params=pltpu.CompilerParams(needs_layout_passes=False,
                                         use_tc_tiling_on_sc=False),
    scratch_types=[
        pltpu.VMEM((NCH, CR), jnp.int32),
        pltpu.VMEM((CR, D), jnp.float32),
        pltpu.VMEM((CR, D), jnp.float32),
        pltpu.VMEM((CR, D), jnp.float32),
        pltpu.VMEM((CR, D), jnp.float32),
        pltpu.VMEM((CR, D), jnp.float32),
        pltpu.VMEM((CR, D), jnp.float32),
        pltpu.VMEM((RPW,), jnp.float32),
        pltpu.SemaphoreType.DMA,
        pltpu.SemaphoreType.DMA,
        pltpu.SemaphoreType.DMA,
    ],
)(_sc_body)


def kernel(subject_embeddings, object_embeddings, relations, relation_table):
    return _sc_call(subject_embeddings, object_embeddings,
                    relations.astype(jnp.int32).reshape(NW, NCH, CR),
                    relation_table)


# speedup vs baseline: 2.3098x; 1.2022x over previous
import functools

import jax
import jax.numpy as jnp
from jax import lax
from jax.experimental import pallas as pl
from jax.experimental.pallas import tpu as pltpu
from jax.experimental.pallas import tpu_sc as plsc

B = 16384
D = 64
NUM_REL = 1000
EPS = 1e-6
NC = 2
NS = 16
NW = NC * NS
RPW = B // NW
NCH = 4
CR = RPW // NCH
CGROUPS = CR // 16


def _sc_body(s_hbm, o_hbm, rel_hbm, tab_hbm, out_hbm,
             idx_v, s0, o0, t0, s1, o1, t1, out_v, sem_t, sem0, sem1):
    cid = lax.axis_index("c")
    sid = lax.axis_index("s")
    wid = sid * NC + cid
    base = wid * RPW

    pltpu.sync_copy(rel_hbm.at[wid], idx_v)

    bufs = [(s0, o0, t0, sem0), (s1, o1, t1, sem1)]

    def issue(c):
        sv, ov, tv, sem = bufs[c % 2]
        rb = base + c * CR
        return (pltpu.async_copy(s_hbm.at[pl.ds(rb, CR)], sv, sem),
                pltpu.async_copy(o_hbm.at[pl.ds(rb, CR)], ov, sem),
                pltpu.async_copy(tab_hbm.at[idx_v.at[c]], tv, sem))

    pending = {0: issue(0), 1: issue(1)}

    lanes = lax.iota(jnp.int32, 16)
    perms = [lanes ^ sh for sh in (8, 4, 2, 1)]

    for c in range(NCH):
        sv_ref, ov_ref, tv_ref, _ = bufs[c % 2]
        for h in pending.pop(c):
            h.wait()

        @plsc.parallel_loop(0, CGROUPS)
        def group(g, sv_ref=sv_ref, ov_ref=ov_ref, tv_ref=tv_ref, c=c):
            acc = jnp.zeros((16,), jnp.float32)
            for j in range(16):
                r = g * 16 + j
                p = None
                for k in range(4):
                    sk = sv_ref[r, pl.ds(k * 16, 16)]
                    ok = ov_ref[r, pl.ds(k * 16, 16)]
                    tk = tv_ref[r, pl.ds(k * 16, 16)]
                    df = sk + tk - ok + EPS
                    sq = df * df
                    p = sq if p is None else p + sq
                for pm in perms:
                    p = p + jnp.take_along_axis(p, pm, axis=0)
                acc = jnp.where(lanes == j, p, acc)
            bits = lax.bitcast_convert_type(acc, jnp.int32)
            y = lax.bitcast_convert_type(
                jnp.int32(0x5F3759DF) - (bits >> 1), jnp.float32)
            for _ in range(3):
                y = y * (1.5 - 0.5 * acc * y * y)
            out_v[pl.ds(c * CR + g * 16, 16)] = acc * y

        if c + 2 < NCH:
            pending[c + 2] = issue(c + 2)

    pltpu.sync_copy(out_v, out_hbm.at[pl.ds(base, RPW)])


_sc_call = functools.partial(
    pl.kernel,
    mesh=plsc.VectorSubcoreMesh(core_axis_name="c", subcore_axis_name="s"),
    out_type=jax.ShapeDtypeStruct((B,), jnp.float32),
    compiler_params=pltpu.CompilerParams(needs_layout_passes=False),
    scratch_types=[
        pltpu.VMEM((NCH, CR), jnp.int32),
        pltpu.VMEM((CR, D), jnp.float32),
        pltpu.VMEM((CR, D), jnp.float32),
        pltpu.VMEM((CR, 128), jnp.float32),
        pltpu.VMEM((CR, D), jnp.float32),
        pltpu.VMEM((CR, D), jnp.float32),
        pltpu.VMEM((CR, 128), jnp.float32),
        pltpu.VMEM((RPW,), jnp.float32),
        pltpu.SemaphoreType.DMA,
        pltpu.SemaphoreType.DMA,
        pltpu.SemaphoreType.DMA,
    ],
)(_sc_body)


def kernel(subject_embeddings, object_embeddings, relations, relation_table):
    tab = jnp.pad(relation_table, ((0, 0), (0, 128 - D)))
    return _sc_call(subject_embeddings, object_embeddings,
                    relations.astype(jnp.int32).reshape(NW, NCH, CR),
                    tab)
